# baseline (speedup 1.0000x reference)
import functools

import jax
import jax.numpy as jnp
from jax import lax
from jax.experimental import pallas as pl
from jax.experimental.pallas import tpu as pltpu
from jax.experimental.pallas import tpu_sc as plsc

_B, _C, _D = 4096, 100, 32
_HLEN, _WLEN, _NTAG = 200, 50, 5
_NC, _NS = 2, 16
_NW = _NC * _NS
_BPW = _B // _NW

_NB_U = 4
_NA = _BPW // _NB_U
_HB = _NB_U * _HLEN
_WB = _NB_U * _WLEN
_NB_I = 64
_NB = _BPW * _C // _NB_I
_TB = _NB_I * _NTAG
_H = 16
_S0 = pl.ds(0, _H)
_S1 = pl.ds(_H, _H)


def _xfer(src, dst, sem, wait):
    if wait:
        pltpu.make_async_copy(src, dst, sem).wait()
    else:
        pltpu.async_copy(src, dst, sem)


_GMAX = 1024


def _gather(table, idx_ref, n, rows_ref, sem, wait):
    if n <= _GMAX:
        _xfer(table.at[idx_ref], rows_ref, sem, wait)
        return
    off = 0
    while off < n:
        m = min(_GMAX, n - off)
        _xfer(table.at[idx_ref.at[pl.ds(off, m)]],
              rows_ref.at[pl.ds(off, m)], sem, wait)
        off += m


def _pipe(n, stage, gather, accum, store):
    stage(0, 0, False)
    stage(0, 0, True)
    gather(0, 0, False)
    stage(1, 1, False)

    def body(t2, carry):
        for s_ in (0, 1):
            t = t2 * 2 + s_
            sb = 1 - s_

            @pl.when(t + 1 < n)
            def _():
                stage(t + 1, sb, True)
                gather(t + 1, sb, False)

            gather(t, s_, True)

            if store is not None:
                @pl.when(t >= 2)
                def _():
                    store(t - 2, s_, True)

            accum(t, s_)

            @pl.when(t + 2 < n)
            def _():
                stage(t + 2, s_, False)

            if store is not None:
                store(t, s_, False)
        return carry

    lax.fori_loop(0, n // 2, body, 0)
    if store is not None:
        store(n - 2, 0, True)
        store(n - 1, 1, True)


def _sc_pool(hist_f, wish_f, cand_f, auth_f, lang_f, tags_f,
             t_hist, t_wish, t_cand, t_auth, t_lang, t_tags):
    mesh = plsc.VectorSubcoreMesh(core_axis_name="c", subcore_axis_name="s")

    @functools.partial(
        pl.kernel,
        out_type=(jax.ShapeDtypeStruct((_B, _D), jnp.float32),
                  jax.ShapeDtypeStruct((_B * _C, _D), jnp.float32)),
        mesh=mesh,
        compiler_params=pltpu.CompilerParams(use_tc_tiling_on_sc=False,
                                             needs_layout_passes=False),
        scratch_types=[
            pltpu.VMEM((_HB,), jnp.int32), pltpu.VMEM((_WB,), jnp.int32),
            pltpu.VMEM((_HB, _D), jnp.float32),
            pltpu.VMEM((_WB, _D), jnp.float32),
            pltpu.VMEM((_HB,), jnp.int32), pltpu.VMEM((_WB,), jnp.int32),
            pltpu.VMEM((_HB, _D), jnp.float32),
            pltpu.VMEM((_WB, _D), jnp.float32),
            pltpu.VMEM((_NB_I,), jnp.int32), pltpu.VMEM((_NB_I,), jnp.int32),
            pltpu.VMEM((_NB_I,), jnp.int32), pltpu.VMEM((_TB,), jnp.int32),
            pltpu.VMEM((_NB_I, _D), jnp.float32),
            pltpu.VMEM((_NB_I, _D), jnp.float32),
            pltpu.VMEM((_NB_I, _D), jnp.float32),
            pltpu.VMEM((_TB, _D), jnp.float32),
            pltpu.VMEM((_NB_I, _D), jnp.float32),
            pltpu.VMEM((_NB_I,), jnp.int32), pltpu.VMEM((_NB_I,), jnp.int32),
            pltpu.VMEM((_NB_I,), jnp.int32), pltpu.VMEM((_TB,), jnp.int32),
            pltpu.VMEM((_NB_I, _D), jnp.float32),
            pltpu.VMEM((_NB_I, _D), jnp.float32),
            pltpu.VMEM((_NB_I, _D), jnp.float32),
            pltpu.VMEM((_TB, _D), jnp.float32),
            pltpu.VMEM((_NB_I, _D), jnp.float32),
            pltpu.VMEM((_BPW, _D), jnp.float32),
            pltpu.VMEM((30, _D), jnp.float32),
            pltpu.SemaphoreType.DMA, pltpu.SemaphoreType.DMA,
            pltpu.SemaphoreType.DMA, pltpu.SemaphoreType.DMA,
            pltpu.SemaphoreType.DMA, pltpu.SemaphoreType.DMA,
        ],
    )
    def k(hist_r, wish_r, cand_r, auth_r, lang_r, tags_r,
          th_r, tw_r, tcand_r, ta_r, tl_r, tt_r,
          u_out, i_out,
          idxh0, idxw0, rowsh0, rowsw0, idxh1, idxw1, rowsh1, rowsw1,
          idxc0, idxa0, idxl0, idxt0, rowsc0, rowsa0, rowsl0, rowst0, obuf0,
          idxc1, idxa1, idxl1, idxt1, rowsc1, rowsa1, rowsl1, rowst1, obuf1,
          u_buf, lang_v, semi0, semi1, semg0, semg1, semo0, semo1):
        idx_h, idx_w = (idxh0, idxh1), (idxw0, idxw1)
        rows_h, rows_w = (rowsh0, rowsh1), (rowsw0, rowsw1)
        idx_c, idx_a = (idxc0, idxc1), (idxa0, idxa1)
        idx_l, idx_t = (idxl0, idxl1), (idxt0, idxt1)
        rows_c, rows_a = (rowsc0, rowsc1), (rowsa0, rowsa1)
        rows_l, rows_t = (rowsl0, rowsl1), (rowst0, rowst1)
        o_buf = (obuf0, obuf1)
        sem_i, sem_g, sem_o = (semi0, semi1), (semg0, semg1), (semo0, semo1)

        wid = lax.axis_index("s") * _NC + lax.axis_index("c")
        b0 = wid * _BPW
        zero = jnp.zeros((_H,), jnp.float32)
        lane16 = jax.lax.iota(jnp.int32, 16)

        pltpu.sync_copy(tl_r, lang_v)

        def a_stage(t, s, wait):
            bb = b0 + t * _NB_U
            _xfer(hist_r.at[pl.ds(pl.multiple_of(bb * _HLEN, 8), _HB)],
                  idx_h[s], sem_i[s], wait)
            _xfer(wish_r.at[pl.ds(pl.multiple_of(bb * _WLEN, 8), _WB)],
                  idx_w[s], sem_i[s], wait)

        def a_gather(t, s, wait):
            _gather(th_r, idx_h[s], _HB, rows_h[s], sem_g[s], wait)
            _gather(tw_r, idx_w[s], _WB, rows_w[s], sem_g[s], wait)

        def a_accum(t, s):
            rh, rw = rows_h[s], rows_w[s]
            for j in range(_NB_U):
                def hbody(r, accs):
                    a0, a1, b0_, b1 = accs
                    p = j * _HLEN + r * 5
                    a0 = a0 + rh[p, _S0]
                    b0_ = b0_ + rh[p, _S1]
                    a1 = a1 + rh[p + 1, _S0]
                    b1 = b1 + rh[p + 1, _S1]
                    a0 = a0 + rh[p + 2, _S0]
                    b0_ = b0_ + rh[p + 2, _S1]
                    a1 = a1 + rh[p + 3, _S0]
                    b1 = b1 + rh[p + 3, _S1]
                    a0 = a0 + rh[p + 4, _S0]
                    b0_ = b0_ + rh[p + 4, _S1]
                    return a0, a1, b0_, b1

                a0, a1, c0, c1 = lax.fori_loop(0, _HLEN // 5, hbody,
                                               (zero, zero, zero, zero))

                def wbody(r, accs):
                    a0, a1, b0_, b1 = accs
                    p = j * _WLEN + r * 5
                    a0 = a0 + rw[p, _S0]
                    b0_ = b0_ + rw[p, _S1]
                    a1 = a1 + rw[p + 1, _S0]
                    b1 = b1 + rw[p + 1, _S1]
                    a0 = a0 + rw[p + 2, _S0]
                    b0_ = b0_ + rw[p + 2, _S1]
                    a1 = a1 + rw[p + 3, _S0]
                    b1 = b1 + rw[p + 3, _S1]
                    a0 = a0 + rw[p + 4, _S0]
                    b0_ = b0_ + rw[p + 4, _S1]
                    return a0, a1, b0_, b1

                w0, w1, x0, x1 = lax.fori_loop(0, _WLEN // 5, wbody,
                                               (zero, zero, zero, zero))
                row = t * _NB_U + j
                u_buf[row, _S0] = ((a0 + a1) * (1.0 / _HLEN)
                                   + (w0 + w1) * (1.0 / _WLEN))
                u_buf[row, _S1] = ((c0 + c1) * (1.0 / _HLEN)
                                   + (x0 + x1) * (1.0 / _WLEN))

        _pipe(_NA, a_stage, a_gather, a_accum, None)
        pltpu.sync_copy(u_buf, u_out.at[pl.ds(pl.multiple_of(b0, 8), _BPW)])

        def b_stage(t, s, wait):
            base = pl.multiple_of(b0 * _C + t * _NB_I, 8)
            tbase = pl.multiple_of((b0 * _C + t * _NB_I) * _NTAG, 8)
            _xfer(cand_r.at[pl.ds(base, _NB_I)], idx_c[s], sem_i[s], wait)
            _xfer(auth_r.at[pl.ds(base, _NB_I)], idx_a[s], sem_i[s], wait)
            _xfer(lang_r.at[pl.ds(base, _NB_I)], idx_l[s], sem_i[s], wait)
            _xfer(tags_r.at[pl.ds(tbase, _TB)], idx_t[s], sem_i[s], wait)

        def b_gather(t, s, wait):
            _gather(tcand_r, idx_c[s], _NB_I, rows_c[s], sem_g[s], wait)
            _gather(ta_r, idx_a[s], _NB_I, rows_a[s], sem_g[s], wait)
            _gather(tt_r, idx_t[s], _TB, rows_t[s], sem_g[s], wait)

        def b_accum(t, s):
            rc, ra, rt, ob = rows_c[s], rows_a[s], rows_t[s], o_buf[s]
            rl = rows_l[s]
            il = idx_l[s]

            @plsc.parallel_loop(0, _NB_I, step=1, unroll=2)
            def _(i):
                p = i * _NTAG
                for hs in (_S0, _S1):
                    v = rc[i, hs] + ra[i, hs]
                    w = (rt[p, hs] + rt[p + 1, hs]) + (rt[p + 2, hs]
                                                       + rt[p + 3, hs])
                    ob[i, hs] = v + (1.0 / _NTAG) * (w + rt[p + 4, hs])

            def lang_add(k, carry):
                i0 = k * 16
                rows16 = il[pl.ds(i0, 16)]
                items16 = lane16 + i0
                for l in range(_D):
                    vals = plsc.load_gather(
                        lang_v, [rows16, jnp.full((16,), l, jnp.int32)])
                    plsc.addupdate_scatter(
                        ob, [items16, jnp.full((16,), l, jnp.int32)], vals)
                return carry

            lax.fori_loop(0, _NB_I // 16, lang_add, 0)

        def b_store(t, s, wait):
            base = pl.multiple_of(b0 * _C + t * _NB_I, 8)
            _xfer(o_buf[s], i_out.at[pl.ds(base, _NB_I)], sem_o[s], wait)

        _pipe(_NB, b_stage, b_gather, b_accum, b_store)

    return k(hist_f, wish_f, cand_f, auth_f, lang_f, tags_f,
             t_hist, t_wish, t_cand, t_auth, t_lang, t_tags)


def _tc_mlp(u_vec, i_vec, dense_f, W1, b1, W2, b2, W3u, W3i, b3,
            W4, b4, W5, b5, W6, b6):
    BB = 32
    R = BB * _C

    def body(u_r, i_r, d_r, w1, c1, w2, c2, w3u, w3i, c3,
             w4, c4, w5, c5, w6, c6, o_r):
        h = jnp.maximum(d_r[...] @ w1[...] + c1[...], 0.0)
        dv = h @ w2[...] + c2[...]
        ifin = i_r[...] + dv
        tu = u_r[...] @ w3u[...]
        ri = lax.broadcasted_iota(jnp.int32, (R, BB), 0) // _C
        ci = lax.broadcasted_iota(jnp.int32, (R, BB), 1)
        e = jnp.where(ri == ci, 1.0, 0.0)
        x = jnp.maximum(ifin @ w3i[...] + e @ tu + c3[...], 0.0)
        x = jnp.maximum(x @ w4[...] + c4[...], 0.0)
        x = jnp.maximum(x @ w5[...] + c5[...], 0.0)
        o_r[...] = x @ w6[...] + c6[...]

    def full(shape):
        return pl.BlockSpec(shape, lambda g: (0,) * len(shape))

    return pl.pallas_call(
        body,
        grid=(_B // BB,),
        in_specs=[
            pl.BlockSpec((BB, _D), lambda g: (g, 0)),
            pl.BlockSpec((R, _D), lambda g: (g, 0)),
            pl.BlockSpec((R, 3), lambda g: (g, 0)),
            full((3, 32)), full((1, 32)), full((32, _D)), full((1, _D)),
            full((_D, 256)), full((_D, 256)), full((1, 256)),
            full((256, 128)), full((1, 128)),
            full((128, 64)), full((1, 64)),
            full((64, 1)), full((1, 1)),
        ],
        out_specs=pl.BlockSpec((R, 1), lambda g: (g, 0)),
        out_shape=jax.ShapeDtypeStruct((_B * _C, 1), jnp.float32),
    )(u_vec, i_vec, dense_f, W1, b1.reshape(1, -1), W2, b2.reshape(1, -1),
      W3u, W3i, b3.reshape(1, -1), W4, b4.reshape(1, -1),
      W5, b5.reshape(1, -1), W6, b6.reshape(1, -1))


def kernel(hist, wish, cand, auth, lang, tags, dense,
           emb_hist, emb_wish, emb_cand, emb_auth, emb_lang, emb_tags,
           W1, b1, W2, b2, W3, b3, W4, b4, W5, b5, W6, b6):
    def i32(x):
        return x.reshape(-1).astype(jnp.int32)

    u_vec, i_vec = _sc_pool(i32(hist), i32(wish), i32(cand), i32(auth),
                            i32(lang), i32(tags),
                            emb_hist, emb_wish, emb_cand, emb_auth,
                            emb_lang, emb_tags)
    scores = _tc_mlp(u_vec, i_vec, dense.reshape(-1, 3),
                     W1, b1, W2, b2, W3[:_D], W3[_D:], b3,
                     W4, b4, W5, b5, W6, b6)
    return scores.reshape(_B, _C)


# speedup vs baseline: 1.2804x; 1.2804x over previous
import functools

import jax
import jax.numpy as jnp
from jax import lax
from jax.experimental import pallas as pl
from jax.experimental.pallas import tpu as pltpu
from jax.experimental.pallas import tpu_sc as plsc

_B, _C, _D = 4096, 100, 32
_HLEN, _WLEN, _NTAG = 200, 50, 5
_NC, _NS = 2, 16
_NW = _NC * _NS
_BPW = _B // _NW

_NB_U = 4
_NA = _BPW // _NB_U
_HB = _NB_U * _HLEN
_WB = _NB_U * _WLEN
_NB_I = 64
_NB = _BPW * _C // _NB_I
_TB = _NB_I * _NTAG
_H = 16
_S0 = pl.ds(0, _H)
_S1 = pl.ds(_H, _H)


def _xfer(src, dst, sem, wait):
    if wait:
        pltpu.make_async_copy(src, dst, sem).wait()
    else:
        pltpu.async_copy(src, dst, sem)


_GMAX = 1024


def _gather(table, idx_ref, n, rows_ref, sem, wait):
    if n <= _GMAX:
        _xfer(table.at[idx_ref], rows_ref, sem, wait)
        return
    off = 0
    while off < n:
        m = min(_GMAX, n - off)
        _xfer(table.at[idx_ref.at[pl.ds(off, m)]],
              rows_ref.at[pl.ds(off, m)], sem, wait)
        off += m


def _pipe(n, stage, gather, accum, store):
    stage(0, 0, False)
    stage(0, 0, True)
    gather(0, 0, False)
    stage(1, 1, False)

    def body(t2, carry):
        for s_ in (0, 1):
            t = t2 * 2 + s_
            sb = 1 - s_

            @pl.when(t + 1 < n)
            def _():
                stage(t + 1, sb, True)
                gather(t + 1, sb, False)

            gather(t, s_, True)

            if store is not None:
                @pl.when(t >= 2)
                def _():
                    store(t - 2, s_, True)

            accum(t, s_)

            @pl.when(t + 2 < n)
            def _():
                stage(t + 2, s_, False)

            if store is not None:
                store(t, s_, False)
        return carry

    lax.fori_loop(0, n // 2, body, 0)
    if store is not None:
        store(n - 2, 0, True)
        store(n - 1, 1, True)


def _sc_pool(hist_f, wish_f, cand_f, auth_f, tags_f,
             t_hist, t_wish, t_cand, t_auth, t_tags):
    mesh = plsc.VectorSubcoreMesh(core_axis_name="c", subcore_axis_name="s")

    @functools.partial(
        pl.kernel,
        out_type=(jax.ShapeDtypeStruct((_B, _D), jnp.float32),
                  jax.ShapeDtypeStruct((_B * _C, _D), jnp.float32)),
        mesh=mesh,
        compiler_params=pltpu.CompilerParams(use_tc_tiling_on_sc=False,
                                             needs_layout_passes=False),
        scratch_types=[
            pltpu.VMEM((_HB,), jnp.int32), pltpu.VMEM((_WB,), jnp.int32),
            pltpu.VMEM((_HB, _D), jnp.float32),
            pltpu.VMEM((_WB, _D), jnp.float32),
            pltpu.VMEM((_HB,), jnp.int32), pltpu.VMEM((_WB,), jnp.int32),
            pltpu.VMEM((_HB, _D), jnp.float32),
            pltpu.VMEM((_WB, _D), jnp.float32),
            pltpu.VMEM((_NB_I,), jnp.int32), pltpu.VMEM((_NB_I,), jnp.int32),
            pltpu.VMEM((_TB,), jnp.int32),
            pltpu.VMEM((_NB_I, _D), jnp.float32),
            pltpu.VMEM((_NB_I, _D), jnp.float32),
            pltpu.VMEM((_TB, _D), jnp.float32),
            pltpu.VMEM((_NB_I, _D), jnp.float32),
            pltpu.VMEM((_NB_I,), jnp.int32), pltpu.VMEM((_NB_I,), jnp.int32),
            pltpu.VMEM((_TB,), jnp.int32),
            pltpu.VMEM((_NB_I, _D), jnp.float32),
            pltpu.VMEM((_NB_I, _D), jnp.float32),
            pltpu.VMEM((_TB, _D), jnp.float32),
            pltpu.VMEM((_NB_I, _D), jnp.float32),
            pltpu.VMEM((_BPW, _D), jnp.float32),
            pltpu.SemaphoreType.DMA, pltpu.SemaphoreType.DMA,
            pltpu.SemaphoreType.DMA, pltpu.SemaphoreType.DMA,
            pltpu.SemaphoreType.DMA, pltpu.SemaphoreType.DMA,
        ],
    )
    def k(hist_r, wish_r, cand_r, auth_r, tags_r,
          th_r, tw_r, tcand_r, ta_r, tt_r,
          u_out, i_out,
          idxh0, idxw0, rowsh0, rowsw0, idxh1, idxw1, rowsh1, rowsw1,
          idxc0, idxa0, idxt0, rowsc0, rowsa0, rowst0, obuf0,
          idxc1, idxa1, idxt1, rowsc1, rowsa1, rowst1, obuf1,
          u_buf, semi0, semi1, semg0, semg1, semo0, semo1):
        idx_h, idx_w = (idxh0, idxh1), (idxw0, idxw1)
        rows_h, rows_w = (rowsh0, rowsh1), (rowsw0, rowsw1)
        idx_c, idx_a = (idxc0, idxc1), (idxa0, idxa1)
        idx_t = (idxt0, idxt1)
        rows_c, rows_a = (rowsc0, rowsc1), (rowsa0, rowsa1)
        rows_t = (rowst0, rowst1)
        o_buf = (obuf0, obuf1)
        sem_i, sem_g, sem_o = (semi0, semi1), (semg0, semg1), (semo0, semo1)

        wid = lax.axis_index("s") * _NC + lax.axis_index("c")
        b0 = wid * _BPW
        zero = jnp.zeros((_H,), jnp.float32)

        def a_stage(t, s, wait):
            bb = b0 + t * _NB_U
            _xfer(hist_r.at[pl.ds(pl.multiple_of(bb * _HLEN, 8), _HB)],
                  idx_h[s], sem_i[s], wait)
            _xfer(wish_r.at[pl.ds(pl.multiple_of(bb * _WLEN, 8), _WB)],
                  idx_w[s], sem_i[s], wait)

        def a_gather(t, s, wait):
            _gather(th_r, idx_h[s], _HB, rows_h[s], sem_g[s], wait)
            _gather(tw_r, idx_w[s], _WB, rows_w[s], sem_g[s], wait)

        def a_accum(t, s):
            rh, rw = rows_h[s], rows_w[s]
            for j in range(_NB_U):
                def hbody(r, accs):
                    a0, a1, b0_, b1 = accs
                    p = j * _HLEN + r * 5
                    a0 = a0 + rh[p, _S0]
                    b0_ = b0_ + rh[p, _S1]
                    a1 = a1 + rh[p + 1, _S0]
                    b1 = b1 + rh[p + 1, _S1]
                    a0 = a0 + rh[p + 2, _S0]
                    b0_ = b0_ + rh[p + 2, _S1]
                    a1 = a1 + rh[p + 3, _S0]
                    b1 = b1 + rh[p + 3, _S1]
                    a0 = a0 + rh[p + 4, _S0]
                    b0_ = b0_ + rh[p + 4, _S1]
                    return a0, a1, b0_, b1

                a0, a1, c0, c1 = lax.fori_loop(0, _HLEN // 5, hbody,
                                               (zero, zero, zero, zero))

                def wbody(r, accs):
                    a0, a1, b0_, b1 = accs
                    p = j * _WLEN + r * 5
                    a0 = a0 + rw[p, _S0]
                    b0_ = b0_ + rw[p, _S1]
                    a1 = a1 + rw[p + 1, _S0]
                    b1 = b1 + rw[p + 1, _S1]
                    a0 = a0 + rw[p + 2, _S0]
                    b0_ = b0_ + rw[p + 2, _S1]
                    a1 = a1 + rw[p + 3, _S0]
                    b1 = b1 + rw[p + 3, _S1]
                    a0 = a0 + rw[p + 4, _S0]
                    b0_ = b0_ + rw[p + 4, _S1]
                    return a0, a1, b0_, b1

                w0, w1, x0, x1 = lax.fori_loop(0, _WLEN // 5, wbody,
                                               (zero, zero, zero, zero))
                row = t * _NB_U + j
                u_buf[row, _S0] = ((a0 + a1) * (1.0 / _HLEN)
                                   + (w0 + w1) * (1.0 / _WLEN))
                u_buf[row, _S1] = ((c0 + c1) * (1.0 / _HLEN)
                                   + (x0 + x1) * (1.0 / _WLEN))

        _pipe(_NA, a_stage, a_gather, a_accum, None)
        pltpu.sync_copy(u_buf, u_out.at[pl.ds(pl.multiple_of(b0, 8), _BPW)])

        def b_stage(t, s, wait):
            base = pl.multiple_of(b0 * _C + t * _NB_I, 8)
            tbase = pl.multiple_of((b0 * _C + t * _NB_I) * _NTAG, 8)
            _xfer(cand_r.at[pl.ds(base, _NB_I)], idx_c[s], sem_i[s], wait)
            _xfer(auth_r.at[pl.ds(base, _NB_I)], idx_a[s], sem_i[s], wait)
            _xfer(tags_r.at[pl.ds(tbase, _TB)], idx_t[s], sem_i[s], wait)

        def b_gather(t, s, wait):
            _gather(tcand_r, idx_c[s], _NB_I, rows_c[s], sem_g[s], wait)
            _gather(ta_r, idx_a[s], _NB_I, rows_a[s], sem_g[s], wait)
            _gather(tt_r, idx_t[s], _TB, rows_t[s], sem_g[s], wait)

        def b_accum(t, s):
            rc, ra, rt, ob = rows_c[s], rows_a[s], rows_t[s], o_buf[s]

            @plsc.parallel_loop(0, _NB_I, step=1, unroll=2)
            def _(i):
                p = i * _NTAG
                for hs in (_S0, _S1):
                    v = rc[i, hs] + ra[i, hs]
                    w = (rt[p, hs] + rt[p + 1, hs]) + (rt[p + 2, hs]
                                                       + rt[p + 3, hs])
                    ob[i, hs] = v + (1.0 / _NTAG) * (w + rt[p + 4, hs])

        def b_store(t, s, wait):
            base = pl.multiple_of(b0 * _C + t * _NB_I, 8)
            _xfer(o_buf[s], i_out.at[pl.ds(base, _NB_I)], sem_o[s], wait)

        _pipe(_NB, b_stage, b_gather, b_accum, b_store)

    return k(hist_f, wish_f, cand_f, auth_f, tags_f,
             t_hist, t_wish, t_cand, t_auth, t_tags)


def _tc_mlp(u_vec, i_vec, dense_f, lang_f, w_lang, W1, b1, W2, b2, W3u, W3i,
            b3, W4, b4, W5, b5, W6, b6):
    BB = 32
    R = BB * _C

    def body(u_r, i_r, d_r, l_r, wl, w1, c1, w2, c2, w3u, w3i, c3,
             w4, c4, w5, c5, w6, c6, o_r):
        h = jnp.maximum(d_r[...] @ w1[...] + c1[...], 0.0)
        dv = h @ w2[...] + c2[...]
        li = lax.broadcasted_iota(jnp.int32, (R, _D), 1)
        eonehot = jnp.where(l_r[...] == li, 1.0, 0.0)
        ifin = i_r[...] + dv + eonehot @ wl[...]
        tu = u_r[...] @ w3u[...]
        ri = lax.broadcasted_iota(jnp.int32, (R, BB), 0) // _C
        ci = lax.broadcasted_iota(jnp.int32, (R, BB), 1)
        e = jnp.where(ri == ci, 1.0, 0.0)
        x = jnp.maximum(ifin @ w3i[...] + e @ tu + c3[...], 0.0)
        x = jnp.maximum(x @ w4[...] + c4[...], 0.0)
        x = jnp.maximum(x @ w5[...] + c5[...], 0.0)
        o_r[...] = x @ w6[...] + c6[...]

    def full(shape):
        return pl.BlockSpec(shape, lambda g: (0,) * len(shape))

    return pl.pallas_call(
        body,
        grid=(_B // BB,),
        in_specs=[
            pl.BlockSpec((BB, _D), lambda g: (g, 0)),
            pl.BlockSpec((R, _D), lambda g: (g, 0)),
            pl.BlockSpec((R, 3), lambda g: (g, 0)),
            pl.BlockSpec((R, 1), lambda g: (g, 0)),
            full((_D, _D)),
            full((3, 32)), full((1, 32)), full((32, _D)), full((1, _D)),
            full((_D, 256)), full((_D, 256)), full((1, 256)),
            full((256, 128)), full((1, 128)),
            full((128, 64)), full((1, 64)),
            full((64, 1)), full((1, 1)),
        ],
        out_specs=pl.BlockSpec((R, 1), lambda g: (g, 0)),
        out_shape=jax.ShapeDtypeStruct((_B * _C, 1), jnp.float32),
    )(u_vec, i_vec, dense_f, lang_f, w_lang,
      W1, b1.reshape(1, -1), W2, b2.reshape(1, -1),
      W3u, W3i, b3.reshape(1, -1), W4, b4.reshape(1, -1),
      W5, b5.reshape(1, -1), W6, b6.reshape(1, -1))


def kernel(hist, wish, cand, auth, lang, tags, dense,
           emb_hist, emb_wish, emb_cand, emb_auth, emb_lang, emb_tags,
           W1, b1, W2, b2, W3, b3, W4, b4, W5, b5, W6, b6):
    def i32(x):
        return x.reshape(-1).astype(jnp.int32)

    u_vec, i_vec = _sc_pool(i32(hist), i32(wish), i32(cand), i32(auth),
                            i32(tags),
                            emb_hist, emb_wish, emb_cand, emb_auth, emb_tags)
    w_lang = jnp.concatenate(
        [emb_lang, jnp.zeros((_D - emb_lang.shape[0], _D), jnp.float32)])
    scores = _tc_mlp(u_vec, i_vec, dense.reshape(-1, 3),
                     lang.reshape(-1, 1).astype(jnp.int32), w_lang,
                     W1, b1, W2, b2, W3[:_D], W3[_D:], b3,
                     W4, b4, W5, b5, W6, b6)
    return scores.reshape(_B, _C)
